# baseline (speedup 1.0000x reference)
import functools

import jax
import jax.numpy as jnp
from jax import lax
from jax.experimental import pallas as pl
from jax.experimental.pallas import tpu as pltpu
from jax.experimental.pallas import tpu_sc as plsc

NC = 2
NS = 16
NW = NC * NS
CHUNK = 128


def _pad_n(n):
    q = NS * CHUNK
    return -(-n // q) * q


def _chunks_per_tile(e):
    c = -(-(-(-e // CHUNK)) // NW)
    return -(-c // 4) * 4


def _make_prop(n, cpt, d):
    n_pad = _pad_n(n)
    rows_per = n_pad // NS
    nfull = rows_per // CHUNK
    half = cpt // 2
    mesh = plsc.VectorSubcoreMesh(core_axis_name="c", subcore_axis_name="s")

    @functools.partial(
        pl.kernel,
        mesh=mesh,
        out_type=jax.ShapeDtypeStruct((NC, n_pad, d), jnp.float32),
        scratch_types=[
            pltpu.VMEM((half, CHUNK), jnp.int32),
            pltpu.VMEM((half, CHUNK), jnp.int32),
            pltpu.VMEM((2, CHUNK, d), jnp.float32),
            pltpu.VMEM_SHARED((n_pad, d), jnp.float32),
            pltpu.SemaphoreType.DMA,
            pltpu.SemaphoreType.DMA,
            pltpu.SemaphoreType.DMA,
            pltpu.SemaphoreType.DMA,
        ],
    )
    def prop(g_hbm, src_hbm, dst_hbm, out_hbm, src_sl, dst_sl, rows_v,
             acc_sh, gsem0, gsem1, ssem0, ssem1):
        cid = lax.axis_index("c")
        sid = lax.axis_index("s")
        wid = sid * NC + cid

        def zrow(i, carry):
            def zlane(j, c2):
                rows_v[0, i, pl.ds(j * 16, 16)] = jnp.zeros((16,),
                                                            jnp.float32)
                return c2
            return lax.fori_loop(0, d // 16, zlane, carry)
        lax.fori_loop(0, CHUNK, zrow, 0)

        base_r = pl.multiple_of(sid * rows_per, CHUNK)

        def zcopy(i, carry):
            pltpu.sync_copy(rows_v.at[0],
                            acc_sh.at[pl.ds(base_r + i * CHUNK, CHUNK), :])
            return carry
        lax.fori_loop(0, nfull, zcopy, 0)
        plsc.subcore_barrier()

        gsems = (gsem0, gsem1)
        ssems = (ssem0, ssem1)

        def gath(i, b):
            pltpu.async_copy(g_hbm.at[src_sl.at[i, pl.ds(0, 64)]],
                             rows_v.at[b, pl.ds(0, 64)], gsems[b])
            pltpu.async_copy(g_hbm.at[src_sl.at[i, pl.ds(64, 64)]],
                             rows_v.at[b, pl.ds(64, 64)], gsems[b])

        def gwait(b):
            pltpu.make_async_copy(g_hbm.at[src_sl.at[0, pl.ds(0, 64)]],
                                  rows_v.at[b, pl.ds(0, 64)],
                                  gsems[b]).wait()
            pltpu.make_async_copy(g_hbm.at[src_sl.at[0, pl.ds(0, 64)]],
                                  rows_v.at[b, pl.ds(0, 64)],
                                  gsems[b]).wait()

        def scat(i, b):
            pltpu.async_copy(rows_v.at[b], acc_sh.at[dst_sl.at[i]],
                             ssems[b], add=True)

        def drain(b):
            pltpu.make_async_copy(rows_v.at[b], acc_sh.at[dst_sl.at[0]],
                                  ssems[b]).wait()

        for h in range(2):
            pltpu.sync_copy(src_hbm.at[wid, pl.ds(h * half, half)], src_sl)
            pltpu.sync_copy(dst_hbm.at[wid, pl.ds(h * half, half)], dst_sl)
            gath(0, 0)

            def pair(p, carry):
                i0 = 2 * p
                gath(i0 + 1, 1)
                gwait(0)
                scat(i0, 0)
                drain(0)
                gath(i0 + 2, 0)
                gwait(1)
                scat(i0 + 1, 1)
                drain(1)
                return carry
            lax.fori_loop(0, half // 2 - 1, pair, 0)
            gath(half - 1, 1)
            gwait(0)
            scat(half - 2, 0)
            drain(0)
            gwait(1)
            scat(half - 1, 1)
            drain(1)
        plsc.subcore_barrier()

        def wcopy(i, carry):
            pltpu.sync_copy(acc_sh.at[pl.ds(base_r + i * CHUNK, CHUNK), :],
                            out_hbm.at[cid,
                                       pl.ds(base_r + i * CHUNK, CHUNK), :])
            return carry
        lax.fori_loop(0, nfull, wcopy, 0)

    return prop


def _make_deg(n, cpt):
    n_pad = _pad_n(n)
    zch = n_pad // NS
    mesh = plsc.VectorSubcoreMesh(core_axis_name="c", subcore_axis_name="s")

    @functools.partial(
        pl.kernel,
        mesh=mesh,
        out_type=jax.ShapeDtypeStruct((NC, n_pad), jnp.float32),
        scratch_types=[
            pltpu.VMEM((cpt, CHUNK), jnp.int32),
            pltpu.VMEM((CHUNK,), jnp.float32),
            pltpu.VMEM((zch,), jnp.float32),
            pltpu.VMEM_SHARED((n_pad,), jnp.float32),
            pltpu.SemaphoreType.DMA,
            pltpu.SemaphoreType.DMA,
        ],
    )
    def degk(dst_hbm, out_hbm, dst_slab, ones_v, zbuf, deg_sh, sem0, sem1):
        cid = lax.axis_index("c")
        sid = lax.axis_index("s")
        wid = sid * NC + cid
        sems = (sem0, sem1)

        pltpu.sync_copy(dst_hbm.at[wid], dst_slab)

        def fill(i, carry):
            zbuf[pl.ds(i * 16, 16)] = jnp.zeros((16,), jnp.float32)
            return carry
        lax.fori_loop(0, zch // 16, fill, 0)

        def fones(i, carry):
            ones_v[pl.ds(i * 16, 16)] = jnp.ones((16,), jnp.float32)
            return carry
        lax.fori_loop(0, CHUNK // 16, fones, 0)

        base_r = pl.multiple_of(sid * zch, CHUNK)
        pltpu.sync_copy(zbuf, deg_sh.at[pl.ds(base_r, zch)])
        plsc.subcore_barrier()

        def body(i, carry):
            h0 = pltpu.async_copy(ones_v, deg_sh.at[dst_slab.at[2 * i]],
                                  sems[0], add=True)
            h1 = pltpu.async_copy(ones_v, deg_sh.at[dst_slab.at[2 * i + 1]],
                                  sems[1], add=True)
            h0.wait()
            h1.wait()
            return carry
        lax.fori_loop(0, cpt // 2, body, 0)
        plsc.subcore_barrier()

        pltpu.sync_copy(deg_sh.at[pl.ds(base_r, zch)],
                        out_hbm.at[cid, pl.ds(base_r, zch)])

    return degk


ROWS = 256


def _dis(degp_ref):
    r0 = pl.program_id(0) * ROWS
    deg = degp_ref[0, pl.ds(r0, ROWS)] + degp_ref[1, pl.ds(r0, ROWS)]
    return jnp.where(deg > 0, lax.rsqrt(deg), 0.0)


def _pre_body(x_ref, wa_ref, ba_ref, degp_ref, h0_ref, g0_ref):
    x = x_ref[...]
    dis = _dis(degp_ref)
    logits = jnp.dot(x, wa_ref[...], preferred_element_type=jnp.float32)
    logits = logits + ba_ref[...]
    m = jnp.max(logits, axis=1, keepdims=True)
    ex = jnp.exp(logits - m)
    sm = ex / jnp.sum(ex, axis=1, keepdims=True)
    h0 = x * sm
    h0_ref[...] = h0
    g0_ref[...] = h0 * dis[:, None]


def _scale_body(ap_ref, degp_ref, g1_ref):
    dis = _dis(degp_ref)
    a = ap_ref[0] + ap_ref[1]
    g1_ref[...] = a * (dis * dis)[:, None]


def _lin_body(y_ref, w_ref, b_ref, p_ref):
    p_ref[...] = (jnp.dot(y_ref[...], w_ref[...],
                          preferred_element_type=jnp.float32) + b_ref[...])


def _mm1_body(p_ref, a0p_ref, a1p_ref, degp_ref, w_ref, out1_ref, g0b_ref):
    dis = _dis(degp_ref)
    h1 = (a0p_ref[0] + a0p_ref[1]) * dis[:, None]
    h2 = (a1p_ref[0] + a1p_ref[1]) * dis[:, None]
    z = (p_ref[...]
         + jnp.dot(h1, w_ref[0], preferred_element_type=jnp.float32)
         + jnp.dot(h2, w_ref[1], preferred_element_type=jnp.float32))
    o = jnp.maximum(z, 0.0)
    out1_ref[...] = o
    g0b_ref[...] = o * dis[:, None]


def _mm2_body(p_ref, a0p_ref, a1p_ref, degp_ref, w_ref, out_ref):
    dis = _dis(degp_ref)
    h1 = (a0p_ref[0] + a0p_ref[1]) * dis[:, None]
    h2 = (a1p_ref[0] + a1p_ref[1]) * dis[:, None]
    z = (p_ref[...]
         + jnp.dot(h1, w_ref[0], preferred_element_type=jnp.float32)
         + jnp.dot(h2, w_ref[1], preferred_element_type=jnp.float32))
    m = jnp.max(z, axis=1, keepdims=True)
    lse = m + jnp.log(jnp.sum(jnp.exp(z - m), axis=1, keepdims=True))
    out_ref[...] = z - lse


def _row_spec(d):
    return pl.BlockSpec((ROWS, d), lambda i: (i, 0))


def _part_spec(d):
    return pl.BlockSpec((NC, ROWS, d), lambda i: (0, i, 0))


def _deg_spec(n_pad):
    return pl.BlockSpec((NC, n_pad), lambda i: (0, 0))


def _full(shape):
    nd = len(shape)
    return pl.BlockSpec(shape, lambda i, _n=nd: (0,) * _n)


def kernel(x, edge_index, Wa, ba, W1, b1, W2, b2):
    n, d_in = x.shape
    e = edge_index.shape[1]
    hid = W1.shape[2]
    d_out = W2.shape[2]
    n_pad = _pad_n(n)
    cpt = _chunks_per_tile(e)
    e_pad = cpt * CHUNK * NW
    pad_idx = n + jnp.arange(e_pad - e, dtype=jnp.int32) % (n_pad - n)
    src = jnp.concatenate([edge_index[0], pad_idx])
    dst = jnp.concatenate([edge_index[1], pad_idx])
    src3 = src.reshape(NW, cpt, CHUNK)
    dst3 = dst.reshape(NW, cpt, CHUNK)
    xp = jnp.pad(x, ((0, n_pad - n), (0, 0)))
    grid = (n_pad // ROWS,)

    degp = _make_deg(n, cpt)(dst3)

    prop = _make_prop(n, cpt, d_in)

    h0, g0 = pl.pallas_call(
        _pre_body,
        grid=grid,
        in_specs=[_row_spec(d_in), _full(Wa.shape), _full((1, d_in)),
                  _deg_spec(n_pad)],
        out_specs=[_row_spec(d_in), _row_spec(d_in)],
        out_shape=[jax.ShapeDtypeStruct((n_pad, d_in), jnp.float32)] * 2,
    )(xp, Wa, ba.reshape(1, -1), degp)

    a0p = prop(g0, src3, dst3)
    p1 = pl.pallas_call(
        _lin_body,
        grid=grid,
        in_specs=[_row_spec(d_in), _full((d_in, hid)), _full((1, hid))],
        out_specs=_row_spec(hid),
        out_shape=jax.ShapeDtypeStruct((n_pad, hid), jnp.float32),
    )(h0, W1[0], b1.reshape(1, -1))
    g1 = pl.pallas_call(
        _scale_body,
        grid=grid,
        in_specs=[_part_spec(d_in), _deg_spec(n_pad)],
        out_specs=_row_spec(d_in),
        out_shape=jax.ShapeDtypeStruct((n_pad, d_in), jnp.float32),
    )(a0p, degp)
    a1p = prop(g1, src3, dst3)

    out1, g0b = pl.pallas_call(
        _mm1_body,
        grid=grid,
        in_specs=[_row_spec(hid), _part_spec(d_in), _part_spec(d_in),
                  _deg_spec(n_pad), _full((2,) + W1.shape[1:])],
        out_specs=[_row_spec(hid), _row_spec(hid)],
        out_shape=[jax.ShapeDtypeStruct((n_pad, hid), jnp.float32)] * 2,
    )(p1, a0p, a1p, degp, W1[1:])

    b0p = prop(g0b, src3, dst3)
    p2 = pl.pallas_call(
        _lin_body,
        grid=grid,
        in_specs=[_row_spec(hid), _full((hid, d_out)), _full((1, d_out))],
        out_specs=_row_spec(d_out),
        out_shape=jax.ShapeDtypeStruct((n_pad, d_out), jnp.float32),
    )(out1, W2[0], b2.reshape(1, -1))
    g1b = pl.pallas_call(
        _scale_body,
        grid=grid,
        in_specs=[_part_spec(hid), _deg_spec(n_pad)],
        out_specs=_row_spec(hid),
        out_shape=jax.ShapeDtypeStruct((n_pad, hid), jnp.float32),
    )(b0p, degp)
    b1p = prop(g1b, src3, dst3)

    out = pl.pallas_call(
        _mm2_body,
        grid=grid,
        in_specs=[_row_spec(d_out), _part_spec(hid), _part_spec(hid),
                  _deg_spec(n_pad), _full((2,) + W2.shape[1:])],
        out_specs=_row_spec(d_out),
        out_shape=jax.ShapeDtypeStruct((n_pad, d_out), jnp.float32),
    )(p2, b0p, b1p, degp, W2[1:])
    return out[:n]


# speedup vs baseline: 1.0053x; 1.0053x over previous
import functools

import jax
import jax.numpy as jnp
from jax import lax
from jax.experimental import pallas as pl
from jax.experimental.pallas import tpu as pltpu
from jax.experimental.pallas import tpu_sc as plsc

NC = 2
NS = 16
NW = NC * NS
CHUNK = 128


def _pad_n(n):
    q = NS * CHUNK
    return -(-n // q) * q


def _chunks_per_tile(e):
    c = -(-(-(-e // CHUNK)) // NW)
    return -(-c // 4) * 4


def _make_prop(n, cpt, d):
    n_pad = _pad_n(n)
    rows_per = n_pad // NS
    nfull = rows_per // CHUNK
    half = cpt // 2
    mesh = plsc.VectorSubcoreMesh(core_axis_name="c", subcore_axis_name="s")

    @functools.partial(
        pl.kernel,
        mesh=mesh,
        out_type=jax.ShapeDtypeStruct((NC, n_pad, d), jnp.float32),
        scratch_types=[
            pltpu.VMEM((half, CHUNK), jnp.int32),
            pltpu.VMEM((half, CHUNK), jnp.int32),
            pltpu.VMEM((2, CHUNK, d), jnp.float32),
            pltpu.VMEM_SHARED((n_pad, d), jnp.float32),
            pltpu.SemaphoreType.DMA,
            pltpu.SemaphoreType.DMA,
            pltpu.SemaphoreType.DMA,
            pltpu.SemaphoreType.DMA,
        ],
    )
    def prop(g_hbm, src_hbm, dst_hbm, out_hbm, src_sl, dst_sl, rows_v,
             acc_sh, gsem0, gsem1, ssem0, ssem1):
        cid = lax.axis_index("c")
        sid = lax.axis_index("s")
        wid = sid * NC + cid

        def zrow(i, carry):
            def zlane(j, c2):
                rows_v[0, i, pl.ds(j * 16, 16)] = jnp.zeros((16,),
                                                            jnp.float32)
                return c2
            return lax.fori_loop(0, d // 16, zlane, carry)
        lax.fori_loop(0, CHUNK, zrow, 0)

        base_r = pl.multiple_of(sid * rows_per, CHUNK)

        def zcopy(i, carry):
            pltpu.sync_copy(rows_v.at[0],
                            acc_sh.at[pl.ds(base_r + i * CHUNK, CHUNK), :])
            return carry
        lax.fori_loop(0, nfull, zcopy, 0)
        plsc.subcore_barrier()

        gsems = (gsem0, gsem1)
        ssems = (ssem0, ssem1)

        def gath(i, b):
            pltpu.async_copy(g_hbm.at[src_sl.at[i, pl.ds(0, 64)]],
                             rows_v.at[b, pl.ds(0, 64)], gsems[b])
            pltpu.async_copy(g_hbm.at[src_sl.at[i, pl.ds(64, 64)]],
                             rows_v.at[b, pl.ds(64, 64)], gsems[b])

        def gwait(b):
            pltpu.make_async_copy(g_hbm.at[src_sl.at[0, pl.ds(0, 64)]],
                                  rows_v.at[b, pl.ds(0, 64)],
                                  gsems[b]).wait()
            pltpu.make_async_copy(g_hbm.at[src_sl.at[0, pl.ds(0, 64)]],
                                  rows_v.at[b, pl.ds(0, 64)],
                                  gsems[b]).wait()

        def scat(i, b):
            pltpu.async_copy(rows_v.at[b], acc_sh.at[dst_sl.at[i]],
                             ssems[b], add=True)

        def drain(b):
            pltpu.make_async_copy(rows_v.at[b], acc_sh.at[dst_sl.at[0]],
                                  ssems[b]).wait()

        for h in range(2):
            pltpu.sync_copy(src_hbm.at[wid, pl.ds(h * half, half)], src_sl)
            pltpu.sync_copy(dst_hbm.at[wid, pl.ds(h * half, half)], dst_sl)
            gath(0, 0)

            def pair(p, carry):
                i0 = 2 * p
                gath(i0 + 1, 1)
                gwait(0)
                scat(i0, 0)
                drain(0)
                gath(i0 + 2, 0)
                gwait(1)
                scat(i0 + 1, 1)
                drain(1)
                return carry
            lax.fori_loop(0, half // 2 - 1, pair, 0)
            gath(half - 1, 1)
            gwait(0)
            scat(half - 2, 0)
            drain(0)
            gwait(1)
            scat(half - 1, 1)
            drain(1)
        plsc.subcore_barrier()

        def wcopy(i, carry):
            pltpu.sync_copy(acc_sh.at[pl.ds(base_r + i * CHUNK, CHUNK), :],
                            out_hbm.at[cid,
                                       pl.ds(base_r + i * CHUNK, CHUNK), :])
            return carry
        lax.fori_loop(0, nfull, wcopy, 0)

    return prop


def _make_deg(n, cpt):
    n_pad = _pad_n(n)
    zch = n_pad // NS
    mesh = plsc.VectorSubcoreMesh(core_axis_name="c", subcore_axis_name="s")

    @functools.partial(
        pl.kernel,
        mesh=mesh,
        out_type=jax.ShapeDtypeStruct((NC, n_pad), jnp.float32),
        scratch_types=[
            pltpu.VMEM((cpt, CHUNK), jnp.int32),
            pltpu.VMEM((CHUNK,), jnp.float32),
            pltpu.VMEM((zch,), jnp.float32),
            pltpu.VMEM_SHARED((n_pad,), jnp.float32),
            pltpu.SemaphoreType.DMA,
            pltpu.SemaphoreType.DMA,
        ],
    )
    def degk(dst_hbm, out_hbm, dst_slab, ones_v, zbuf, deg_sh, sem0, sem1):
        cid = lax.axis_index("c")
        sid = lax.axis_index("s")
        wid = sid * NC + cid
        sems = (sem0, sem1)

        pltpu.sync_copy(dst_hbm.at[wid], dst_slab)

        def fill(i, carry):
            zbuf[pl.ds(i * 16, 16)] = jnp.zeros((16,), jnp.float32)
            return carry
        lax.fori_loop(0, zch // 16, fill, 0)

        def fones(i, carry):
            ones_v[pl.ds(i * 16, 16)] = jnp.ones((16,), jnp.float32)
            return carry
        lax.fori_loop(0, CHUNK // 16, fones, 0)

        base_r = pl.multiple_of(sid * zch, CHUNK)
        pltpu.sync_copy(zbuf, deg_sh.at[pl.ds(base_r, zch)])
        plsc.subcore_barrier()

        def body(i, carry):
            h0 = pltpu.async_copy(ones_v, deg_sh.at[dst_slab.at[2 * i]],
                                  sems[0], add=True)
            h1 = pltpu.async_copy(ones_v, deg_sh.at[dst_slab.at[2 * i + 1]],
                                  sems[1], add=True)
            h0.wait()
            h1.wait()
            return carry
        lax.fori_loop(0, cpt // 2, body, 0)
        plsc.subcore_barrier()

        pltpu.sync_copy(deg_sh.at[pl.ds(base_r, zch)],
                        out_hbm.at[cid, pl.ds(base_r, zch)])

    return degk


ROWS = 256


def _dis(degp_ref):
    r0 = pl.program_id(0) * ROWS
    deg = degp_ref[0, pl.ds(r0, ROWS)] + degp_ref[1, pl.ds(r0, ROWS)]
    return jnp.where(deg > 0, lax.rsqrt(deg), 0.0)


def _pre_body(x_ref, wa_ref, ba_ref, degp_ref, h0_ref, g0_ref):
    x = x_ref[...]
    dis = _dis(degp_ref)
    logits = jnp.dot(x, wa_ref[...], preferred_element_type=jnp.float32)
    logits = logits + ba_ref[...]
    m = jnp.max(logits, axis=1, keepdims=True)
    ex = jnp.exp(logits - m)
    sm = ex / jnp.sum(ex, axis=1, keepdims=True)
    h0 = x * sm
    h0_ref[...] = h0
    g0_ref[...] = h0 * dis[:, None]


def _scale_body(ap_ref, degp_ref, g1_ref):
    dis = _dis(degp_ref)
    a = ap_ref[0] + ap_ref[1]
    g1_ref[...] = a * (dis * dis)[:, None]


def _mm1_body(h0_ref, a0p_ref, a1p_ref, degp_ref, w_ref, b_ref,
              out1_ref, g0b_ref):
    dis = _dis(degp_ref)
    h1 = (a0p_ref[0] + a0p_ref[1]) * dis[:, None]
    h2 = (a1p_ref[0] + a1p_ref[1]) * dis[:, None]
    z = (jnp.dot(h0_ref[...], w_ref[0], preferred_element_type=jnp.float32)
         + jnp.dot(h1, w_ref[1], preferred_element_type=jnp.float32)
         + jnp.dot(h2, w_ref[2], preferred_element_type=jnp.float32)
         + b_ref[...])
    o = jnp.maximum(z, 0.0)
    out1_ref[...] = o
    g0b_ref[...] = o * dis[:, None]


def _mm2_body(h0_ref, a0p_ref, a1p_ref, degp_ref, w_ref, b_ref, out_ref):
    dis = _dis(degp_ref)
    h1 = (a0p_ref[0] + a0p_ref[1]) * dis[:, None]
    h2 = (a1p_ref[0] + a1p_ref[1]) * dis[:, None]
    z = (jnp.dot(h0_ref[...], w_ref[0], preferred_element_type=jnp.float32)
         + jnp.dot(h1, w_ref[1], preferred_element_type=jnp.float32)
         + jnp.dot(h2, w_ref[2], preferred_element_type=jnp.float32)
         + b_ref[...])
    m = jnp.max(z, axis=1, keepdims=True)
    lse = m + jnp.log(jnp.sum(jnp.exp(z - m), axis=1, keepdims=True))
    out_ref[...] = z - lse


def _row_spec(d):
    return pl.BlockSpec((ROWS, d), lambda i: (i, 0))


def _part_spec(d):
    return pl.BlockSpec((NC, ROWS, d), lambda i: (0, i, 0))


def _deg_spec(n_pad):
    return pl.BlockSpec((NC, n_pad), lambda i: (0, 0))


def _full(shape):
    nd = len(shape)
    return pl.BlockSpec(shape, lambda i, _n=nd: (0,) * _n)


def kernel(x, edge_index, Wa, ba, W1, b1, W2, b2):
    n, d_in = x.shape
    e = edge_index.shape[1]
    hid = W1.shape[2]
    d_out = W2.shape[2]
    n_pad = _pad_n(n)
    cpt = _chunks_per_tile(e)
    e_pad = cpt * CHUNK * NW
    pad_idx = n + jnp.arange(e_pad - e, dtype=jnp.int32) % (n_pad - n)
    src = jnp.concatenate([edge_index[0], pad_idx])
    dst = jnp.concatenate([edge_index[1], pad_idx])
    src3 = src.reshape(NW, cpt, CHUNK)
    dst3 = dst.reshape(NW, cpt, CHUNK)
    xp = jnp.pad(x, ((0, n_pad - n), (0, 0)))
    grid = (n_pad // ROWS,)

    degp = _make_deg(n, cpt)(dst3)

    prop = _make_prop(n, cpt, d_in)

    h0, g0 = pl.pallas_call(
        _pre_body,
        grid=grid,
        in_specs=[_row_spec(d_in), _full(Wa.shape), _full((1, d_in)),
                  _deg_spec(n_pad)],
        out_specs=[_row_spec(d_in), _row_spec(d_in)],
        out_shape=[jax.ShapeDtypeStruct((n_pad, d_in), jnp.float32)] * 2,
    )(xp, Wa, ba.reshape(1, -1), degp)

    a0p = prop(g0, src3, dst3)
    g1 = pl.pallas_call(
        _scale_body,
        grid=grid,
        in_specs=[_part_spec(d_in), _deg_spec(n_pad)],
        out_specs=_row_spec(d_in),
        out_shape=jax.ShapeDtypeStruct((n_pad, d_in), jnp.float32),
    )(a0p, degp)
    a1p = prop(g1, src3, dst3)

    out1, g0b = pl.pallas_call(
        _mm1_body,
        grid=grid,
        in_specs=[_row_spec(d_in), _part_spec(d_in), _part_spec(d_in),
                  _deg_spec(n_pad), _full(W1.shape), _full((1, hid))],
        out_specs=[_row_spec(hid), _row_spec(hid)],
        out_shape=[jax.ShapeDtypeStruct((n_pad, hid), jnp.float32)] * 2,
    )(h0, a0p, a1p, degp, W1, b1.reshape(1, -1))

    b0p = prop(g0b, src3, dst3)
    g1b = pl.pallas_call(
        _scale_body,
        grid=grid,
        in_specs=[_part_spec(hid), _deg_spec(n_pad)],
        out_specs=_row_spec(hid),
        out_shape=jax.ShapeDtypeStruct((n_pad, hid), jnp.float32),
    )(b0p, degp)
    b1p = prop(g1b, src3, dst3)

    out = pl.pallas_call(
        _mm2_body,
        grid=grid,
        in_specs=[_row_spec(hid), _part_spec(hid), _part_spec(hid),
                  _deg_spec(n_pad), _full(W2.shape), _full((1, d_out))],
        out_specs=_row_spec(d_out),
        out_shape=jax.ShapeDtypeStruct((n_pad, d_out), jnp.float32),
    )(out1, b0p, b1p, degp, W2, b2.reshape(1, -1))
    return out[:n]
